# baseline (speedup 1.0000x reference)
import functools

import jax
import jax.numpy as jnp
from jax import lax
from jax.experimental import pallas as pl
from jax.experimental.pallas import tpu as pltpu
from jax.experimental.pallas import tpu_sc as plsc

NY, NX = 496, 432
B = 4
S = NY * NX
TOTAL = B * S
P = 40000
C = 64
EB = 6

NW = 32
SLOTS = 26880
TOTAL_PAD = NW * SLOTS
CV_ROWS = TOTAL_PAD + NW
SEGSZ = 1280
NSEG = SLOTS // SEGSZ
DCH = 64
NCH_MAX = SEGSZ // DCH
CCH = 4000
NCC = P // CCH
ZROWS = 320
NZ = SLOTS // ZROWS
BIG = 0x40000000

_mesh = plsc.VectorSubcoreMesh(core_axis_name="c", subcore_axis_name="s")


@functools.partial(
    pl.kernel,
    out_type=jax.ShapeDtypeStruct((CV_ROWS, C), jnp.float32),
    mesh=_mesh,
    compiler_params=pltpu.CompilerParams(
        needs_layout_passes=False, use_tc_tiling_on_sc=False
    ),
    scratch_types=[
        pltpu.VMEM((SLOTS,), jnp.int32),
        pltpu.VMEM((CCH,), jnp.int32),
        pltpu.VMEM((CCH,), jnp.int32),
        pltpu.VMEM((CCH,), jnp.int32),
        pltpu.VMEM((SEGSZ,), jnp.int32),
        pltpu.VMEM((SEGSZ,), jnp.int32),
        pltpu.VMEM((NCH_MAX, DCH), jnp.int32),
        pltpu.VMEM((NCH_MAX, DCH), jnp.int32),
        pltpu.VMEM((DCH, C), jnp.float32),
        pltpu.VMEM((ZROWS, C), jnp.float32),
        pltpu.SemaphoreType.DMA,
        pltpu.SemaphoreType.DMA,
    ],
)
def _sc_build_canvas(bcol_hbm, ycol_hbm, xcol_hbm, vf_hbm, cv_hbm, map_ref,
                     bc_ref, yc_ref, xc_ref, wp_ref,
                     ws_ref, wp2_ref, ws2_ref, rows_ref, zero_ref,
                     zsem, dsem):
    wid = lax.axis_index("s") * 2 + lax.axis_index("c")
    base = wid * SLOTS
    dump_row = TOTAL_PAD + wid
    iota = lax.broadcasted_iota(jnp.int32, (16,), 0)

    zf16 = jnp.zeros((16,), jnp.float32)
    def _zfill(i, _):
        r = i >> 2
        cc = (i & 3) * 16
        zero_ref[r, pl.ds(cc, 16)] = zf16
        return 0
    lax.fori_loop(0, ZROWS * (C // 16), _zfill, 0)

    def _zstart(z, _):
        pltpu.make_async_copy(
            zero_ref, cv_hbm.at[pl.ds(base + z * ZROWS, ZROWS)], zsem
        ).start()
        return 0
    lax.fori_loop(0, NZ, _zstart, 0)

    neg1 = jnp.full((16,), -1, jnp.int32)
    def _minit(i, _):
        map_ref[pl.ds(i * 16, 16)] = neg1
        return 0
    lax.fori_loop(0, SLOTS // 16, _minit, 0)

    for cchunk in range(NCC):
        pltpu.sync_copy(bcol_hbm.at[pl.ds(cchunk * CCH, CCH)], bc_ref)
        pltpu.sync_copy(ycol_hbm.at[pl.ds(cchunk * CCH, CCH)], yc_ref)
        pltpu.sync_copy(xcol_hbm.at[pl.ds(cchunk * CCH, CCH)], xc_ref)

        def _mbody(j, _, base_p=cchunk * CCH):
            off = j * 16
            bb = bc_ref[pl.ds(off, 16)]
            yy = yc_ref[pl.ds(off, 16)]
            xx = xc_ref[pl.ds(off, 16)]
            pp = base_p + off + iota
            flat = bb * S + yy * NX + xx
            flat = jnp.where(bb < B, flat, BIG)
            inr = (flat >= base) & (flat < base + SLOTS)
            local = jnp.where(inr, flat - base, 0)
            old = plsc.load_gather(map_ref, [local], mask=inr)
            need = inr & (old < pp)

            def _wcond(need):
                return jnp.sum(need.astype(jnp.int32)) > 0

            def _wbody(need):
                plsc.store_scatter(map_ref, [local], pp, mask=need)
                q = plsc.load_gather(map_ref, [local], mask=inr)
                return inr & (q < pp)

            lax.while_loop(_wcond, _wbody, need)
            return 0
        lax.fori_loop(0, CCH // 16, _mbody, 0)

    def _zdrain(z, _):
        pltpu.make_async_copy(
            zero_ref, cv_hbm.at[pl.ds(base, ZROWS)], zsem
        ).wait()
        return 0
    lax.fori_loop(0, NZ, _zdrain, 0)

    zero16 = jnp.zeros((16,), jnp.int32)
    dump16 = jnp.full((16,), dump_row, jnp.int32)
    for seg in range(NSEG):
        sbase = seg * SEGSZ

        def _prefill(j, _):
            wp_ref[pl.ds(j * 16, 16)] = zero16
            ws_ref[pl.ds(j * 16, 16)] = dump16
            return 0
        lax.fori_loop(0, SEGSZ // 16, _prefill, 0)

        def _compact(j, cnt, sbase=sbase):
            v = map_ref[pl.ds(sbase + j * 16, 16)]
            m = v >= 0
            rowg = base + sbase + j * 16 + iota
            plsc.store_compressed(wp_ref.at[pl.ds(cnt, 16)], v, mask=m)
            plsc.store_compressed(ws_ref.at[pl.ds(cnt, 16)], rowg, mask=m)
            return cnt + jnp.sum(m.astype(jnp.int32))
        cnt = lax.fori_loop(0, SEGSZ // 16, _compact, 0)

        ndma = (cnt + (DCH - 1)) // DCH

        def _tochunk(k, _):
            for t in range(DCH // 16):
                wp2_ref[k, pl.ds(t * 16, 16)] = wp_ref[pl.ds(k * DCH + t * 16, 16)]
                ws2_ref[k, pl.ds(t * 16, 16)] = ws_ref[pl.ds(k * DCH + t * 16, 16)]
            return 0
        lax.fori_loop(0, ndma, _tochunk, 0)

        def _dma(k, _):
            pltpu.async_copy(vf_hbm.at[wp2_ref.at[k]], rows_ref, dsem).wait()
            pltpu.async_copy(rows_ref, cv_hbm.at[ws2_ref.at[k]], dsem).wait()
            return 0
        lax.fori_loop(0, ndma, _dma, 0)


_GRID_I = 62
PBS = S // _GRID_I


def _conv_block(cv_ref, bev_ref, wc_ref, wb_ref, bias_ref):
    yt = lax.dot_general(
        wc_ref[...], cv_ref[...], (((1,), (1,)), ((), ())),
        preferred_element_type=jnp.float32,
    )
    yt += lax.dot_general(
        wb_ref[...], bev_ref[0], (((1,), (0,)), ((), ())),
        preferred_element_type=jnp.float32,
    )
    return yt + bias_ref[...]


def _stats_body(cv_ref, bev_ref, wc_ref, wb_ref, bias_ref, out_ref):
    yt = _conv_block(cv_ref, bev_ref, wc_ref, wb_ref, bias_ref)
    s1 = jnp.sum(yt, axis=1)
    s2 = jnp.sum(yt * yt, axis=1)
    st = jnp.concatenate([s1[None, :], s2[None, :]], axis=0)
    first = (pl.program_id(0) == 0) & (pl.program_id(1) == 0)

    @pl.when(first)
    def _():
        out_ref[...] = st

    @pl.when(jnp.logical_not(first))
    def _():
        out_ref[...] += st


def _final_body(cv_ref, bev_ref, wc_ref, wb_ref, bias_ref, ss_ref, out_ref):
    yt = _conv_block(cv_ref, bev_ref, wc_ref, wb_ref, bias_ref)
    scale = ss_ref[0][:, None]
    shift = ss_ref[1][:, None]
    out_ref[0] = jnp.maximum(yt * scale + shift, 0.0)


def _small_specs():
    return [
        pl.BlockSpec((C, C), lambda bi, i: (0, 0)),
        pl.BlockSpec((C, EB), lambda bi, i: (0, 0)),
        pl.BlockSpec((C, 1), lambda bi, i: (0, 0)),
    ]


def _data_specs():
    return [
        pl.BlockSpec((PBS, C), lambda bi, i: (bi * _GRID_I + i, 0)),
        pl.BlockSpec((1, EB, PBS), lambda bi, i: (bi, 0, i)),
    ]


def kernel(voxel_features, coors, batch_size, bev_features, W, b, gamma, beta):
    del batch_size
    cv = _sc_build_canvas(coors[:, 0], coors[:, 2], coors[:, 3], voxel_features)

    wc = W[:, :C]
    wb = W[:, C:]
    bias = b[:, None]
    bev_r = bev_features.reshape(B, EB, S)

    stats = pl.pallas_call(
        _stats_body,
        grid=(B, _GRID_I),
        in_specs=_data_specs() + _small_specs(),
        out_specs=pl.BlockSpec((2, C), lambda bi, i: (0, 0)),
        out_shape=jax.ShapeDtypeStruct((2, C), jnp.float32),
    )(cv, bev_r, wc, wb, bias)

    n = float(TOTAL)
    mean = stats[0] / n
    var = stats[1] / n - mean * mean
    scale = gamma * lax.rsqrt(var + 1e-5)
    shift = beta - mean * scale
    ss = jnp.concatenate([scale[None, :], shift[None, :]], axis=0)

    out = pl.pallas_call(
        _final_body,
        grid=(B, _GRID_I),
        in_specs=_data_specs() + _small_specs()
        + [pl.BlockSpec((2, C), lambda bi, i: (0, 0))],
        out_specs=pl.BlockSpec((1, C, PBS), lambda bi, i: (bi, 0, i)),
        out_shape=jax.ShapeDtypeStruct((B, C, S), jnp.float32),
    )(cv, bev_r, wc, wb, bias, ss)

    return out.reshape(B, C, NY, NX)


# speedup vs baseline: 1.1428x; 1.1428x over previous
import functools

import jax
import jax.numpy as jnp
from jax import lax
from jax.experimental import pallas as pl
from jax.experimental.pallas import tpu as pltpu
from jax.experimental.pallas import tpu_sc as plsc

NY, NX = 496, 432
B = 4
S = NY * NX
HALF = 2 * S
P = 40000
C = 64
EB = 6

NW = 32
SLOTS = 13440
ROWS_PAD = NW * SLOTS
CV_ROWS = ROWS_PAD + NW
MSLOTS = 2 * SLOTS
SEGSZ = 1344
NSEG = SLOTS // SEGSZ
DCH = 64
NCH_MAX = SEGSZ // DCH
PPAD = P + 16
ZIDX = P
CCH = 4000
NCC = P // CCH
ZROWS = 160
NZ = SLOTS // ZROWS
BIG = 0x40000000

_mesh = plsc.VectorSubcoreMesh(core_axis_name="c", subcore_axis_name="s")


@functools.partial(
    pl.kernel,
    out_type=jax.ShapeDtypeStruct((CV_ROWS, 2 * C), jnp.float32),
    mesh=_mesh,
    compiler_params=pltpu.CompilerParams(
        needs_layout_passes=False, use_tc_tiling_on_sc=False
    ),
    scratch_types=[
        pltpu.VMEM((MSLOTS,), jnp.int32),
        pltpu.VMEM((CCH,), jnp.int32),
        pltpu.VMEM((CCH,), jnp.int32),
        pltpu.VMEM((CCH,), jnp.int32),
        pltpu.VMEM((SEGSZ,), jnp.int32),
        pltpu.VMEM((SEGSZ,), jnp.int32),
        pltpu.VMEM((SEGSZ,), jnp.int32),
        pltpu.VMEM((NCH_MAX, DCH), jnp.int32),
        pltpu.VMEM((NCH_MAX, DCH), jnp.int32),
        pltpu.VMEM((NCH_MAX, DCH), jnp.int32),
        pltpu.VMEM((DCH, C), jnp.float32),
        pltpu.VMEM((DCH, C), jnp.float32),
        pltpu.VMEM((DCH, 2 * C), jnp.float32),
        pltpu.VMEM((ZROWS, 2 * C), jnp.float32),
        pltpu.SemaphoreType.DMA,
        pltpu.SemaphoreType.DMA,
    ],
)
def _sc_build_canvas(bcol_hbm, ycol_hbm, xcol_hbm, vf_hbm, cv_hbm, map_ref,
                     bc_ref, yc_ref, xc_ref, w0_ref, w1_ref,
                     wr_ref, w0c_ref, w1c_ref, wrc_ref, r0_ref, r1_ref,
                     rows_ref, zero_ref, zsem, dsem):
    wid = lax.axis_index("s") * 2 + lax.axis_index("c")
    base = wid * SLOTS
    dump_row = ROWS_PAD + wid
    iota = lax.broadcasted_iota(jnp.int32, (16,), 0)

    zf16 = jnp.zeros((16,), jnp.float32)
    def _zfill(i, _):
        r = i >> 3
        cc = (i & 7) * 16
        zero_ref[r, pl.ds(cc, 16)] = zf16
        return 0
    lax.fori_loop(0, ZROWS * (2 * C // 16), _zfill, 0)

    def _zstart(z, _):
        pltpu.make_async_copy(
            zero_ref, cv_hbm.at[pl.ds(base + z * ZROWS, ZROWS)], zsem
        ).start()
        return 0
    lax.fori_loop(0, NZ, _zstart, 0)

    neg1 = jnp.full((16,), -1, jnp.int32)
    def _minit(i, _):
        map_ref[pl.ds(i * 16, 16)] = neg1
        return 0
    lax.fori_loop(0, MSLOTS // 16, _minit, 0)

    for cchunk in range(NCC):
        pltpu.sync_copy(bcol_hbm.at[pl.ds(cchunk * CCH, CCH)], bc_ref)
        pltpu.sync_copy(ycol_hbm.at[pl.ds(cchunk * CCH, CCH)], yc_ref)
        pltpu.sync_copy(xcol_hbm.at[pl.ds(cchunk * CCH, CCH)], xc_ref)

        def _mbody(j, _, base_p=cchunk * CCH):
            off = j * 16
            bb = bc_ref[pl.ds(off, 16)]
            yy = yc_ref[pl.ds(off, 16)]
            xx = xc_ref[pl.ds(off, 16)]
            pp = base_p + off + iota
            flat = bb * S + yy * NX + xx
            flat = jnp.where(bb < B, flat, BIG)
            l0 = flat - base
            in0 = (l0 >= 0) & (l0 < SLOTS) & (flat < HALF)
            l1 = flat - (HALF + base - SLOTS)
            in1 = (l1 >= SLOTS) & (l1 < MSLOTS)
            inr = in0 | in1
            local = jnp.where(in0, l0, l1)
            local = jnp.where(inr, local, 0)
            old = plsc.load_gather(map_ref, [local], mask=inr)
            need = inr & (old < pp)

            def _wcond(need):
                return jnp.sum(need.astype(jnp.int32)) > 0

            def _wbody(need):
                plsc.store_scatter(map_ref, [local], pp, mask=need)
                q = plsc.load_gather(map_ref, [local], mask=inr)
                return inr & (q < pp)

            lax.while_loop(_wcond, _wbody, need)
            return 0
        lax.fori_loop(0, CCH // 16, _mbody, 0)

    def _zdrain(z, _):
        pltpu.make_async_copy(
            zero_ref, cv_hbm.at[pl.ds(base, ZROWS)], zsem
        ).wait()
        return 0
    lax.fori_loop(0, NZ, _zdrain, 0)

    zidx16 = jnp.full((16,), ZIDX, jnp.int32)
    dump16 = jnp.full((16,), dump_row, jnp.int32)
    for seg in range(NSEG):
        sbase = seg * SEGSZ
        rowbase = base + sbase

        def _prefill(j, _):
            w0_ref[pl.ds(j * 16, 16)] = zidx16
            w1_ref[pl.ds(j * 16, 16)] = zidx16
            wr_ref[pl.ds(j * 16, 16)] = dump16
            return 0
        lax.fori_loop(0, SEGSZ // 16, _prefill, 0)

        def _compact(j, cnt, sbase=sbase, rowbase=rowbase):
            v0 = map_ref[pl.ds(sbase + j * 16, 16)]
            v1 = map_ref[pl.ds(SLOTS + sbase + j * 16, 16)]
            m = (v0 >= 0) | (v1 >= 0)
            rowg = rowbase + j * 16 + iota
            g0 = jnp.where(v0 >= 0, v0, ZIDX)
            g1 = jnp.where(v1 >= 0, v1, ZIDX)
            plsc.store_compressed(w0_ref.at[pl.ds(cnt, 16)], g0, mask=m)
            plsc.store_compressed(w1_ref.at[pl.ds(cnt, 16)], g1, mask=m)
            plsc.store_compressed(wr_ref.at[pl.ds(cnt, 16)], rowg, mask=m)
            return cnt + jnp.sum(m.astype(jnp.int32))
        cnt = lax.fori_loop(0, SEGSZ // 16, _compact, 0)

        ndma = (cnt + (DCH - 1)) // DCH

        def _tochunk(k, _):
            for t in range(DCH // 16):
                w0c_ref[k, pl.ds(t * 16, 16)] = w0_ref[pl.ds(k * DCH + t * 16, 16)]
                w1c_ref[k, pl.ds(t * 16, 16)] = w1_ref[pl.ds(k * DCH + t * 16, 16)]
                wrc_ref[k, pl.ds(t * 16, 16)] = wr_ref[pl.ds(k * DCH + t * 16, 16)]
            return 0
        lax.fori_loop(0, ndma, _tochunk, 0)

        def _dma(k, _):
            c0 = pltpu.async_copy(vf_hbm.at[w0c_ref.at[k]], r0_ref, dsem)
            c1 = pltpu.async_copy(vf_hbm.at[w1c_ref.at[k]], r1_ref, dsem)
            c0.wait()
            c1.wait()

            def _merge(r, _):
                for t in range(C // 16):
                    rows_ref[r, pl.ds(t * 16, 16)] = r0_ref[r, pl.ds(t * 16, 16)]
                    rows_ref[r, pl.ds(C + t * 16, 16)] = r1_ref[r, pl.ds(t * 16, 16)]
                return 0
            lax.fori_loop(0, DCH, _merge, 0)
            pltpu.async_copy(rows_ref, cv_hbm.at[wrc_ref.at[k]], dsem).wait()
            return 0
        lax.fori_loop(0, ndma, _dma, 0)


_GRID_I = 62
PBS = S // _GRID_I


def _conv_block(cv_half, bev_blk, wc_ref, wb_ref, bias_ref):
    yt = lax.dot_general(
        wc_ref[...], cv_half, (((1,), (1,)), ((), ())),
        preferred_element_type=jnp.float32,
    )
    yt += lax.dot_general(
        wb_ref[...], bev_blk, (((1,), (0,)), ((), ())),
        preferred_element_type=jnp.float32,
    )
    return yt + bias_ref[...]


def _stats_body(cv_ref, bev_ref, wc_ref, wb_ref, bias_ref, out_ref):
    yt0 = _conv_block(cv_ref[:, :C], bev_ref[0, 0], wc_ref, wb_ref, bias_ref)
    yt1 = _conv_block(cv_ref[:, C:], bev_ref[1, 0], wc_ref, wb_ref, bias_ref)
    s1 = jnp.sum(yt0, axis=1) + jnp.sum(yt1, axis=1)
    s2 = jnp.sum(yt0 * yt0, axis=1) + jnp.sum(yt1 * yt1, axis=1)
    st = jnp.concatenate([s1[None, :], s2[None, :]], axis=0)
    first = (pl.program_id(0) == 0) & (pl.program_id(1) == 0)

    @pl.when(first)
    def _():
        out_ref[...] = st

    @pl.when(jnp.logical_not(first))
    def _():
        out_ref[...] += st


def _final_body(cv_ref, bev_ref, wc_ref, wb_ref, bias_ref, ss_ref, out_ref):
    scale = ss_ref[0][:, None]
    shift = ss_ref[1][:, None]
    yt0 = _conv_block(cv_ref[:, :C], bev_ref[0, 0], wc_ref, wb_ref, bias_ref)
    out_ref[0, 0] = jnp.maximum(yt0 * scale + shift, 0.0)
    yt1 = _conv_block(cv_ref[:, C:], bev_ref[1, 0], wc_ref, wb_ref, bias_ref)
    out_ref[1, 0] = jnp.maximum(yt1 * scale + shift, 0.0)


def _small_specs():
    return [
        pl.BlockSpec((C, C), lambda bi, i: (0, 0)),
        pl.BlockSpec((C, EB), lambda bi, i: (0, 0)),
        pl.BlockSpec((C, 1), lambda bi, i: (0, 0)),
    ]


def _data_specs():
    return [
        pl.BlockSpec((PBS, 2 * C), lambda bi, i: (bi * _GRID_I + i, 0)),
        pl.BlockSpec((2, 1, EB, PBS), lambda bi, i: (0, bi, 0, i)),
    ]


def kernel(voxel_features, coors, batch_size, bev_features, W, b, gamma, beta):
    del batch_size
    vfp = jnp.concatenate(
        [voxel_features, jnp.zeros((PPAD - P, C), voxel_features.dtype)], axis=0
    )
    cv = _sc_build_canvas(coors[:, 0], coors[:, 2], coors[:, 3], vfp)

    wc = W[:, :C]
    wb = W[:, C:]
    bias = b[:, None]
    bev_r = bev_features.reshape(2, 2, EB, S)

    stats = pl.pallas_call(
        _stats_body,
        grid=(2, _GRID_I),
        in_specs=_data_specs() + _small_specs(),
        out_specs=pl.BlockSpec((2, C), lambda bi, i: (0, 0)),
        out_shape=jax.ShapeDtypeStruct((2, C), jnp.float32),
    )(cv, bev_r, wc, wb, bias)

    n = float(B * S)
    mean = stats[0] / n
    var = stats[1] / n - mean * mean
    scale = gamma * lax.rsqrt(var + 1e-5)
    shift = beta - mean * scale
    ss = jnp.concatenate([scale[None, :], shift[None, :]], axis=0)

    out = pl.pallas_call(
        _final_body,
        grid=(2, _GRID_I),
        in_specs=_data_specs() + _small_specs()
        + [pl.BlockSpec((2, C), lambda bi, i: (0, 0))],
        out_specs=pl.BlockSpec((2, 1, C, PBS), lambda bi, i: (0, bi, 0, i)),
        out_shape=jax.ShapeDtypeStruct((2, 2, C, S), jnp.float32),
    )(cv, bev_r, wc, wb, bias, ss)

    return out.reshape(B, C, NY, NX)


# speedup vs baseline: 1.3789x; 1.2066x over previous
import functools

import jax
import jax.numpy as jnp
from jax import lax
from jax.experimental import pallas as pl
from jax.experimental.pallas import tpu as pltpu
from jax.experimental.pallas import tpu_sc as plsc

NY, NX = 496, 432
NXP = 512
B = 4
S = NY * NX
PS = NY * NXP
HALF = 2 * PS
P = 40000
C = 64
EB = 6

NW = 32
SLOTS = HALF // NW
ROWS_PAD = NW * SLOTS
CV_ROWS = ROWS_PAD + NW
MSLOTS = 2 * SLOTS
SEGSZ = 1984
NSEG = SLOTS // SEGSZ
DCH = 64
NCH_MAX = SEGSZ // DCH
PPAD = P + 16
ZIDX = P
CCH = 4000
NCC = P // CCH
ZROWS = 128
NZ = SLOTS // ZROWS
BIG = 0x40000000
NPADTOT = B * NY * (NXP - NX)

_mesh = plsc.VectorSubcoreMesh(core_axis_name="c", subcore_axis_name="s")


@functools.partial(
    pl.kernel,
    out_type=jax.ShapeDtypeStruct((CV_ROWS, 2 * C), jnp.float32),
    mesh=_mesh,
    compiler_params=pltpu.CompilerParams(
        needs_layout_passes=False, use_tc_tiling_on_sc=False
    ),
    scratch_types=[
        pltpu.VMEM((MSLOTS,), jnp.int32),
        pltpu.VMEM((CCH,), jnp.int32),
        pltpu.VMEM((CCH,), jnp.int32),
        pltpu.VMEM((CCH,), jnp.int32),
        pltpu.VMEM((SEGSZ,), jnp.int32),
        pltpu.VMEM((SEGSZ,), jnp.int32),
        pltpu.VMEM((SEGSZ,), jnp.int32),
        pltpu.VMEM((NCH_MAX, DCH), jnp.int32),
        pltpu.VMEM((NCH_MAX, DCH), jnp.int32),
        pltpu.VMEM((NCH_MAX, DCH), jnp.int32),
        pltpu.VMEM((DCH, C), jnp.float32),
        pltpu.VMEM((DCH, C), jnp.float32),
        pltpu.VMEM((DCH, 2 * C), jnp.float32),
        pltpu.VMEM((ZROWS, 2 * C), jnp.float32),
        pltpu.SemaphoreType.DMA,
        pltpu.SemaphoreType.DMA,
    ],
)
def _sc_build_canvas(bcol_hbm, ycol_hbm, xcol_hbm, vf_hbm, cv_hbm, map_ref,
                     bc_ref, yc_ref, xc_ref, w0_ref, w1_ref,
                     wr_ref, w0c_ref, w1c_ref, wrc_ref, r0_ref, r1_ref,
                     rows_ref, zero_ref, zsem, dsem):
    wid = lax.axis_index("s") * 2 + lax.axis_index("c")
    base = wid * SLOTS
    dump_row = ROWS_PAD + wid
    iota = lax.broadcasted_iota(jnp.int32, (16,), 0)

    zf16 = jnp.zeros((16,), jnp.float32)
    def _zfill(i, _):
        r = i >> 3
        cc = (i & 7) * 16
        zero_ref[r, pl.ds(cc, 16)] = zf16
        return 0
    lax.fori_loop(0, ZROWS * (2 * C // 16), _zfill, 0)

    def _zstart(z, _):
        pltpu.make_async_copy(
            zero_ref, cv_hbm.at[pl.ds(base + z * ZROWS, ZROWS)], zsem
        ).start()
        return 0
    lax.fori_loop(0, NZ, _zstart, 0)

    neg1 = jnp.full((16,), -1, jnp.int32)
    def _minit(i, _):
        map_ref[pl.ds(i * 16, 16)] = neg1
        return 0
    lax.fori_loop(0, MSLOTS // 16, _minit, 0)

    for cchunk in range(NCC):
        pltpu.sync_copy(bcol_hbm.at[pl.ds(cchunk * CCH, CCH)], bc_ref)
        pltpu.sync_copy(ycol_hbm.at[pl.ds(cchunk * CCH, CCH)], yc_ref)
        pltpu.sync_copy(xcol_hbm.at[pl.ds(cchunk * CCH, CCH)], xc_ref)

        def _mbody(j, _, base_p=cchunk * CCH):
            off = j * 16
            bb = bc_ref[pl.ds(off, 16)]
            yy = yc_ref[pl.ds(off, 16)]
            xx = xc_ref[pl.ds(off, 16)]
            pp = base_p + off + iota
            flat = bb * PS + yy * NXP + xx
            flat = jnp.where(bb < B, flat, BIG)
            l0 = flat - base
            in0 = (l0 >= 0) & (l0 < SLOTS) & (flat < HALF)
            l1 = flat - (HALF + base - SLOTS)
            in1 = (l1 >= SLOTS) & (l1 < MSLOTS)
            inr = in0 | in1
            local = jnp.where(in0, l0, l1)
            local = jnp.where(inr, local, 0)
            old = plsc.load_gather(map_ref, [local], mask=inr)
            need = inr & (old < pp)

            def _wcond(need):
                return jnp.sum(need.astype(jnp.int32)) > 0

            def _wbody(need):
                plsc.store_scatter(map_ref, [local], pp, mask=need)
                q = plsc.load_gather(map_ref, [local], mask=inr)
                return inr & (q < pp)

            lax.while_loop(_wcond, _wbody, need)
            return 0
        lax.fori_loop(0, CCH // 16, _mbody, 0)

    def _zdrain(z, _):
        pltpu.make_async_copy(
            zero_ref, cv_hbm.at[pl.ds(base, ZROWS)], zsem
        ).wait()
        return 0
    lax.fori_loop(0, NZ, _zdrain, 0)

    zidx16 = jnp.full((16,), ZIDX, jnp.int32)
    dump16 = jnp.full((16,), dump_row, jnp.int32)
    for seg in range(NSEG):
        sbase = seg * SEGSZ
        rowbase = base + sbase

        def _prefill(j, _):
            w0_ref[pl.ds(j * 16, 16)] = zidx16
            w1_ref[pl.ds(j * 16, 16)] = zidx16
            wr_ref[pl.ds(j * 16, 16)] = dump16
            return 0
        lax.fori_loop(0, SEGSZ // 16, _prefill, 0)

        def _compact(j, cnt, sbase=sbase, rowbase=rowbase):
            v0 = map_ref[pl.ds(sbase + j * 16, 16)]
            v1 = map_ref[pl.ds(SLOTS + sbase + j * 16, 16)]
            m = (v0 >= 0) | (v1 >= 0)
            rowg = rowbase + j * 16 + iota
            g0 = jnp.where(v0 >= 0, v0, ZIDX)
            g1 = jnp.where(v1 >= 0, v1, ZIDX)
            plsc.store_compressed(w0_ref.at[pl.ds(cnt, 16)], g0, mask=m)
            plsc.store_compressed(w1_ref.at[pl.ds(cnt, 16)], g1, mask=m)
            plsc.store_compressed(wr_ref.at[pl.ds(cnt, 16)], rowg, mask=m)
            return cnt + jnp.sum(m.astype(jnp.int32))
        cnt = lax.fori_loop(0, SEGSZ // 16, _compact, 0)

        ndma = (cnt + (DCH - 1)) // DCH

        def _tochunk(k, _):
            for t in range(DCH // 16):
                w0c_ref[k, pl.ds(t * 16, 16)] = w0_ref[pl.ds(k * DCH + t * 16, 16)]
                w1c_ref[k, pl.ds(t * 16, 16)] = w1_ref[pl.ds(k * DCH + t * 16, 16)]
                wrc_ref[k, pl.ds(t * 16, 16)] = wr_ref[pl.ds(k * DCH + t * 16, 16)]
            return 0
        lax.fori_loop(0, ndma, _tochunk, 0)

        def _dma(k, _):
            c0 = pltpu.async_copy(vf_hbm.at[w0c_ref.at[k]], r0_ref, dsem)
            c1 = pltpu.async_copy(vf_hbm.at[w1c_ref.at[k]], r1_ref, dsem)
            c0.wait()
            c1.wait()

            def _merge(r, _):
                for t in range(C // 16):
                    rows_ref[r, pl.ds(t * 16, 16)] = r0_ref[r, pl.ds(t * 16, 16)]
                    rows_ref[r, pl.ds(C + t * 16, 16)] = r1_ref[r, pl.ds(t * 16, 16)]
                return 0
            lax.fori_loop(0, DCH, _merge, 0)
            pltpu.async_copy(rows_ref, cv_hbm.at[wrc_ref.at[k]], dsem).wait()
            return 0
        lax.fori_loop(0, ndma, _dma, 0)


_GRID_I = 62
PBS = PS // _GRID_I


def _conv_block(cv_half, bev_blk, wc_ref, wb_ref, bias_ref):
    yt = lax.dot_general(
        wc_ref[...], cv_half, (((1,), (1,)), ((), ())),
        preferred_element_type=jnp.float32,
    )
    yt += lax.dot_general(
        wb_ref[...], bev_blk, (((1,), (0,)), ((), ())),
        preferred_element_type=jnp.float32,
    )
    return yt + bias_ref[...]


def _stats_body(cv_ref, bev_ref, wc_ref, wb_ref, bias_ref, out_ref):
    yt0 = _conv_block(cv_ref[:, :C], bev_ref[0, 0], wc_ref, wb_ref, bias_ref)
    yt1 = _conv_block(cv_ref[:, C:], bev_ref[1, 0], wc_ref, wb_ref, bias_ref)
    s1 = jnp.sum(yt0, axis=1) + jnp.sum(yt1, axis=1)
    s2 = jnp.sum(yt0 * yt0, axis=1) + jnp.sum(yt1 * yt1, axis=1)
    st = jnp.concatenate([s1[None, :], s2[None, :]], axis=0)
    first = (pl.program_id(0) == 0) & (pl.program_id(1) == 0)

    @pl.when(first)
    def _():
        out_ref[...] = st

    @pl.when(jnp.logical_not(first))
    def _():
        out_ref[...] += st


def _final_body(cv_ref, bev_ref, wc_ref, wb_ref, bias_ref, ss_ref, out_ref):
    scale = ss_ref[0][:, None]
    shift = ss_ref[1][:, None]
    yt0 = _conv_block(cv_ref[:, :C], bev_ref[0, 0], wc_ref, wb_ref, bias_ref)
    r0 = jnp.maximum(yt0 * scale + shift, 0.0)
    out_ref[0, 0] = r0.reshape(C, PBS // NXP, NXP)[:, :, :NX]
    yt1 = _conv_block(cv_ref[:, C:], bev_ref[1, 0], wc_ref, wb_ref, bias_ref)
    r1 = jnp.maximum(yt1 * scale + shift, 0.0)
    out_ref[1, 0] = r1.reshape(C, PBS // NXP, NXP)[:, :, :NX]


def _small_specs():
    return [
        pl.BlockSpec((C, C), lambda bi, i: (0, 0)),
        pl.BlockSpec((C, EB), lambda bi, i: (0, 0)),
        pl.BlockSpec((C, 1), lambda bi, i: (0, 0)),
    ]


def _data_specs():
    return [
        pl.BlockSpec((PBS, 2 * C), lambda bi, i: (bi * _GRID_I + i, 0)),
        pl.BlockSpec((2, 1, EB, PBS), lambda bi, i: (0, bi, 0, i)),
    ]


def kernel(voxel_features, coors, batch_size, bev_features, W, b, gamma, beta):
    del batch_size
    vfp = jnp.concatenate(
        [voxel_features, jnp.zeros((PPAD - P, C), voxel_features.dtype)], axis=0
    )
    cv = _sc_build_canvas(coors[:, 0], coors[:, 2], coors[:, 3], vfp)

    wc = W[:, :C]
    wb = W[:, C:]
    bias = b[:, None]
    bev_p = jnp.pad(
        bev_features.reshape(B, EB, NY, NX), ((0, 0), (0, 0), (0, 0), (0, NXP - NX))
    )
    bev_r = bev_p.reshape(2, 2, EB, PS)

    stats = pl.pallas_call(
        _stats_body,
        grid=(2, _GRID_I),
        in_specs=_data_specs() + _small_specs(),
        out_specs=pl.BlockSpec((2, C), lambda bi, i: (0, 0)),
        out_shape=jax.ShapeDtypeStruct((2, C), jnp.float32),
    )(cv, bev_r, wc, wb, bias)

    n = float(B * S)
    s1 = stats[0] - NPADTOT * b
    s2 = stats[1] - NPADTOT * b * b
    mean = s1 / n
    var = s2 / n - mean * mean
    scale = gamma * lax.rsqrt(var + 1e-5)
    shift = beta - mean * scale
    ss = jnp.concatenate([scale[None, :], shift[None, :]], axis=0)

    out = pl.pallas_call(
        _final_body,
        grid=(2, _GRID_I),
        in_specs=_data_specs() + _small_specs()
        + [pl.BlockSpec((2, C), lambda bi, i: (0, 0))],
        out_specs=pl.BlockSpec(
            (2, 1, C, PBS // NXP, NX), lambda bi, i: (0, bi, 0, i, 0)
        ),
        out_shape=jax.ShapeDtypeStruct((2, 2, C, NY, NX), jnp.float32),
    )(cv, bev_r, wc, wb, bias, ss)

    return out.reshape(B, C, NY, NX)
